# baseline (device time: 135899 ns/iter reference)
import jax
import jax.numpy as jnp
from jax import lax
from jax.experimental import pallas as pl
from jax.experimental.pallas import tpu as pltpu

N_DEV = 8


def _gelu(y):
    c = 0.7978845608028654
    return 0.5 * y * (1.0 + jnp.tanh(c * (y + 0.044715 * y * y * y)))


def kernel(x, w_mat):
    m, k = x.shape
    _, n = w_mat.shape
    chunk = m // N_DEV

    def body(x_ref, w_ref, out_ref, recv_buf, send_sem, recv_sem, credit_sem):
        my = lax.axis_index("i")
        left = lax.rem(my + N_DEV - 1, N_DEV)
        right = lax.rem(my + 1, N_DEV)

        barrier = pltpu.get_barrier_semaphore()
        for nbr in (left, right):
            pl.semaphore_signal(
                barrier, inc=1,
                device_id=(nbr,), device_id_type=pl.DeviceIdType.MESH,
            )
        pl.semaphore_wait(barrier, 2)

        out_ref[:, :] = jnp.dot(
            x_ref[:, :], w_ref[:, :], preferred_element_type=jnp.float32
        )

        n_hops = 2 * (N_DEV - 1)

        def one_hop(hop, send_idx, recv_idx, is_reduce):
            if hop > 0:
                pl.semaphore_wait(credit_sem, 1)
            rdma = pltpu.make_async_remote_copy(
                src_ref=out_ref.at[pl.ds(send_idx * chunk, chunk), :],
                dst_ref=recv_buf,
                send_sem=send_sem,
                recv_sem=recv_sem,
                device_id=(right,),
                device_id_type=pl.DeviceIdType.MESH,
            )
            rdma.start()
            rdma.wait()
            sl = pl.ds(recv_idx * chunk, chunk)
            if is_reduce:
                out_ref[sl, :] = out_ref[sl, :] + recv_buf[:, :]
            else:
                out_ref[sl, :] = recv_buf[:, :]
            if hop < n_hops - 1:
                pl.semaphore_signal(
                    credit_sem, inc=1,
                    device_id=(left,), device_id_type=pl.DeviceIdType.MESH,
                )

        for s in range(N_DEV - 1):
            send_idx = lax.rem(my - s + 2 * N_DEV, N_DEV)
            recv_idx = lax.rem(my - s - 1 + 2 * N_DEV, N_DEV)
            one_hop(s, send_idx, recv_idx, is_reduce=True)

        for s in range(N_DEV - 1):
            send_idx = lax.rem(my + 1 - s + 2 * N_DEV, N_DEV)
            recv_idx = lax.rem(my - s + 2 * N_DEV, N_DEV)
            one_hop(N_DEV - 1 + s, send_idx, recv_idx, is_reduce=False)

        out_ref[:, :] = _gelu(out_ref[:, :])

    return pl.pallas_call(
        body,
        out_shape=jax.ShapeDtypeStruct((m, n), jnp.float32),
        in_specs=[
            pl.BlockSpec(memory_space=pltpu.VMEM),
            pl.BlockSpec(memory_space=pltpu.VMEM),
        ],
        out_specs=pl.BlockSpec(memory_space=pltpu.VMEM),
        scratch_shapes=[
            pltpu.VMEM((chunk, n), jnp.float32),
            pltpu.SemaphoreType.DMA,
            pltpu.SemaphoreType.DMA,
            pltpu.SemaphoreType.REGULAR,
        ],
        compiler_params=pltpu.CompilerParams(collective_id=0),
    )(x, w_mat)


# device time: 78979 ns/iter; 1.7207x vs baseline; 1.7207x over previous
import jax
import jax.numpy as jnp
from jax import lax
from jax.experimental import pallas as pl
from jax.experimental.pallas import tpu as pltpu

N_DEV = 8


def _gelu(y):
    c = 0.7978845608028654
    return 0.5 * y * (1.0 + jnp.tanh(c * (y + 0.044715 * y * y * y)))


def kernel(x, w_mat):
    m, k = x.shape
    _, n = w_mat.shape
    chunk = m // N_DEV
    half = n // 2
    n_hops = 2 * (N_DEV - 1)

    def body(x_ref, w_ref, out_ref,
             recv_cw, recv_ccw,
             send_sem_cw, recv_sems_cw,
             send_sem_ccw, recv_sems_ccw,
             credit_cw, credit_ccw):
        my = lax.axis_index("i")
        left = lax.rem(my + N_DEV - 1, N_DEV)
        right = lax.rem(my + 1, N_DEV)

        barrier = pltpu.get_barrier_semaphore()
        for nbr in (left, right):
            pl.semaphore_signal(
                barrier, inc=1,
                device_id=(nbr,), device_id_type=pl.DeviceIdType.MESH,
            )
        pl.semaphore_wait(barrier, 2)

        out_ref[:, :] = jnp.dot(
            x_ref[:, :], w_ref[:, :], preferred_element_type=jnp.float32
        )

        def row(idx):
            return pl.ds(idx * chunk, chunk)

        cw_cols = pl.ds(0, half)
        ccw_cols = pl.ds(half, half)

        prev = [None, None]

        for h in range(n_hops):
            slot = h % 2
            if h < N_DEV - 1:
                s = h
                cw_send = lax.rem(my - s + 2 * N_DEV, N_DEV)
                cw_recv = lax.rem(my - s - 1 + 2 * N_DEV, N_DEV)
                ccw_send = lax.rem(my + s, N_DEV)
                ccw_recv = lax.rem(my + s + 1, N_DEV)
            else:
                s = h - (N_DEV - 1)
                cw_send = lax.rem(my + 1 - s + 2 * N_DEV, N_DEV)
                cw_recv = lax.rem(my - s + 2 * N_DEV, N_DEV)
                ccw_send = lax.rem(my - 1 + s + 2 * N_DEV, N_DEV)
                ccw_recv = lax.rem(my + s, N_DEV)

            if h >= 2:
                pl.semaphore_wait(credit_cw, 1)
                pl.semaphore_wait(credit_ccw, 1)
            if h >= 1:
                prev[0].wait_send()
                prev[1].wait_send()

            rdma_cw = pltpu.make_async_remote_copy(
                src_ref=out_ref.at[row(cw_send), cw_cols],
                dst_ref=recv_cw.at[slot],
                send_sem=send_sem_cw,
                recv_sem=recv_sems_cw.at[slot],
                device_id=(right,),
                device_id_type=pl.DeviceIdType.MESH,
            )
            rdma_ccw = pltpu.make_async_remote_copy(
                src_ref=out_ref.at[row(ccw_send), ccw_cols],
                dst_ref=recv_ccw.at[slot],
                send_sem=send_sem_ccw,
                recv_sem=recv_sems_ccw.at[slot],
                device_id=(left,),
                device_id_type=pl.DeviceIdType.MESH,
            )
            rdma_cw.start()
            rdma_ccw.start()

            rdma_cw.wait_recv()
            if h < N_DEV - 1:
                out_ref[row(cw_recv), cw_cols] = (
                    out_ref[row(cw_recv), cw_cols] + recv_cw[slot]
                )
            else:
                out_ref[row(cw_recv), cw_cols] = recv_cw[slot]
            if h <= n_hops - 3:
                pl.semaphore_signal(
                    credit_cw, inc=1,
                    device_id=(left,), device_id_type=pl.DeviceIdType.MESH,
                )

            rdma_ccw.wait_recv()
            if h < N_DEV - 1:
                out_ref[row(ccw_recv), ccw_cols] = (
                    out_ref[row(ccw_recv), ccw_cols] + recv_ccw[slot]
                )
            else:
                out_ref[row(ccw_recv), ccw_cols] = recv_ccw[slot]
            if h <= n_hops - 3:
                pl.semaphore_signal(
                    credit_ccw, inc=1,
                    device_id=(right,), device_id_type=pl.DeviceIdType.MESH,
                )

            prev = [rdma_cw, rdma_ccw]

            if h == N_DEV - 2:
                red_cw = lax.rem(my + 1, N_DEV)
                red_ccw = lax.rem(my + N_DEV - 1, N_DEV)
                out_ref[row(red_cw), cw_cols] = _gelu(
                    out_ref[row(red_cw), cw_cols]
                )
                out_ref[row(red_ccw), ccw_cols] = _gelu(
                    out_ref[row(red_ccw), ccw_cols]
                )

        prev[0].wait_send()
        prev[1].wait_send()

    return pl.pallas_call(
        body,
        out_shape=jax.ShapeDtypeStruct((m, n), jnp.float32),
        in_specs=[
            pl.BlockSpec(memory_space=pltpu.VMEM),
            pl.BlockSpec(memory_space=pltpu.VMEM),
        ],
        out_specs=pl.BlockSpec(memory_space=pltpu.VMEM),
        scratch_shapes=[
            pltpu.VMEM((2, chunk, half), jnp.float32),
            pltpu.VMEM((2, chunk, half), jnp.float32),
            pltpu.SemaphoreType.DMA,
            pltpu.SemaphoreType.DMA((2,)),
            pltpu.SemaphoreType.DMA,
            pltpu.SemaphoreType.DMA((2,)),
            pltpu.SemaphoreType.REGULAR,
            pltpu.SemaphoreType.REGULAR,
        ],
        compiler_params=pltpu.CompilerParams(collective_id=0),
    )(x, w_mat)


# device time: 54909 ns/iter; 2.4750x vs baseline; 1.4384x over previous
import jax
import jax.numpy as jnp
from jax import lax
from jax.experimental import pallas as pl
from jax.experimental.pallas import tpu as pltpu

N_DEV = 8
N_SUB = 2


def _gelu(y):
    c = 0.7978845608028654
    return 0.5 * y * (1.0 + jnp.tanh(c * (y + 0.044715 * y * y * y)))


def kernel(x, w_mat):
    m, k = x.shape
    _, n = w_mat.shape
    chunk = m // N_DEV
    sub = chunk // N_SUB
    half = n // 2
    n_hops = 2 * (N_DEV - 1)

    def body(x_ref, w_ref, out_ref,
             recv_cw, recv_ccw,
             send_sems_cw, recv_sems_cw,
             send_sems_ccw, recv_sems_ccw,
             credits_cw, credits_ccw):
        my = lax.axis_index("i")
        left = lax.rem(my + N_DEV - 1, N_DEV)
        right = lax.rem(my + 1, N_DEV)

        barrier = pltpu.get_barrier_semaphore()
        for nbr in (left, right):
            pl.semaphore_signal(
                barrier, inc=1,
                device_id=(nbr,), device_id_type=pl.DeviceIdType.MESH,
            )
        pl.semaphore_wait(barrier, 2)

        out_ref[:, :] = jnp.dot(
            x_ref[:, :], w_ref[:, :], preferred_element_type=jnp.float32
        )

        cols = (pl.ds(0, half), pl.ds(half, half))
        recv_buf = (recv_cw, recv_ccw)
        send_sems = (send_sems_cw, send_sems_ccw)
        recv_sems = (recv_sems_cw, recv_sems_ccw)
        credits = (credits_cw, credits_ccw)
        peer = (right, left)
        upstream = (left, right)

        def send_idx_of(dirn, h):
            if h < N_DEV - 1:
                d = -h if dirn == 0 else h
            else:
                s = h - (N_DEV - 1)
                d = 1 - s if dirn == 0 else s - 1
            return lax.rem(my + d + 2 * N_DEV, N_DEV)

        def recv_idx_of(dirn, h):
            if h < N_DEV - 1:
                d = -h - 1 if dirn == 0 else h + 1
            else:
                s = h - (N_DEV - 1)
                d = -s if dirn == 0 else s
            return lax.rem(my + d + 2 * N_DEV, N_DEV)

        def mk(dirn, h, s):
            slot = h % 2
            rows = pl.ds(send_idx_of(dirn, h) * chunk + s * sub, sub)
            src = out_ref.at[rows, cols[dirn]]
            if h < N_DEV - 1:
                dst = recv_buf[dirn].at[slot, pl.ds(s * sub, sub), :]
            else:
                dst = out_ref.at[rows, cols[dirn]]
            return pltpu.make_async_remote_copy(
                src_ref=src,
                dst_ref=dst,
                send_sem=send_sems[dirn].at[s],
                recv_sem=recv_sems[dirn].at[slot, s],
                device_id=(peer[dirn],),
                device_id_type=pl.DeviceIdType.MESH,
            )

        inflight = {}
        for s in range(N_SUB):
            for dirn in (0, 1):
                d = mk(dirn, 0, s)
                d.start()
                inflight[(dirn, s)] = d

        for h in range(n_hops):
            for s in range(N_SUB):
                for dirn in (0, 1):
                    d = inflight[(dirn, s)]
                    d.wait_recv()
                    if h < N_DEV - 1:
                        slot = h % 2
                        rows = pl.ds(recv_idx_of(dirn, h) * chunk + s * sub, sub)
                        acc = (out_ref[rows, cols[dirn]]
                               + recv_buf[dirn][slot, pl.ds(s * sub, sub), :])
                        if h == N_DEV - 2:
                            acc = _gelu(acc)
                        out_ref[rows, cols[dirn]] = acc
                    if h <= n_hops - 3:
                        pl.semaphore_signal(
                            credits[dirn].at[s], inc=1,
                            device_id=(upstream[dirn],),
                            device_id_type=pl.DeviceIdType.MESH,
                        )
                    if h + 1 < n_hops:
                        if h + 1 >= 2:
                            pl.semaphore_wait(credits[dirn].at[s], 1)
                        d.wait_send()
                        d2 = mk(dirn, h + 1, s)
                        d2.start()
                        inflight[(dirn, s)] = d2
                    else:
                        d.wait_send()

    return pl.pallas_call(
        body,
        out_shape=jax.ShapeDtypeStruct((m, n), jnp.float32),
        in_specs=[
            pl.BlockSpec(memory_space=pltpu.VMEM),
            pl.BlockSpec(memory_space=pltpu.VMEM),
        ],
        out_specs=pl.BlockSpec(memory_space=pltpu.VMEM),
        scratch_shapes=[
            pltpu.VMEM((2, chunk, half), jnp.float32),
            pltpu.VMEM((2, chunk, half), jnp.float32),
            pltpu.SemaphoreType.DMA((N_SUB,)),
            pltpu.SemaphoreType.DMA((2, N_SUB)),
            pltpu.SemaphoreType.DMA((N_SUB,)),
            pltpu.SemaphoreType.DMA((2, N_SUB)),
            pltpu.SemaphoreType.REGULAR((N_SUB,)),
            pltpu.SemaphoreType.REGULAR((N_SUB,)),
        ],
        compiler_params=pltpu.CompilerParams(collective_id=0),
    )(x, w_mat)


# device time: 51134 ns/iter; 2.6577x vs baseline; 1.0738x over previous
import jax
import jax.numpy as jnp
from jax import lax
from jax.experimental import pallas as pl
from jax.experimental.pallas import tpu as pltpu

N_DEV = 8
N_SUB = 4


def _gelu(y):
    c = 0.7978845608028654
    return 0.5 * y * (1.0 + jnp.tanh(c * (y + 0.044715 * y * y * y)))


def kernel(x, w_mat):
    m, k = x.shape
    _, n = w_mat.shape
    chunk = m // N_DEV
    sub = chunk // N_SUB
    half = n // 2
    n_hops = 2 * (N_DEV - 1)

    def body(x_ref, w_ref, out_ref,
             recv_cw, recv_ccw,
             send_sems_cw, recv_sems_cw,
             send_sems_ccw, recv_sems_ccw,
             credits_cw, credits_ccw):
        my = lax.axis_index("i")
        left = lax.rem(my + N_DEV - 1, N_DEV)
        right = lax.rem(my + 1, N_DEV)

        barrier = pltpu.get_barrier_semaphore()
        for nbr in (left, right):
            pl.semaphore_signal(
                barrier, inc=1,
                device_id=(nbr,), device_id_type=pl.DeviceIdType.MESH,
            )
        pl.semaphore_wait(barrier, 2)

        out_ref[:, :] = jnp.dot(
            x_ref[:, :], w_ref[:, :], preferred_element_type=jnp.float32
        )

        cols = (pl.ds(0, half), pl.ds(half, half))
        recv_buf = (recv_cw, recv_ccw)
        send_sems = (send_sems_cw, send_sems_ccw)
        recv_sems = (recv_sems_cw, recv_sems_ccw)
        credits = (credits_cw, credits_ccw)
        peer = (right, left)
        upstream = (left, right)

        def send_idx_of(dirn, h):
            if h < N_DEV - 1:
                d = -h if dirn == 0 else h
            else:
                s = h - (N_DEV - 1)
                d = 1 - s if dirn == 0 else s - 1
            return lax.rem(my + d + 2 * N_DEV, N_DEV)

        def recv_idx_of(dirn, h):
            if h < N_DEV - 1:
                d = -h - 1 if dirn == 0 else h + 1
            else:
                s = h - (N_DEV - 1)
                d = -s if dirn == 0 else s
            return lax.rem(my + d + 2 * N_DEV, N_DEV)

        def mk(dirn, h, s):
            slot = h % 2
            rows = pl.ds(send_idx_of(dirn, h) * chunk + s * sub, sub)
            src = out_ref.at[rows, cols[dirn]]
            if h < N_DEV - 1:
                dst = recv_buf[dirn].at[slot, pl.ds(s * sub, sub), :]
            else:
                dst = out_ref.at[rows, cols[dirn]]
            return pltpu.make_async_remote_copy(
                src_ref=src,
                dst_ref=dst,
                send_sem=send_sems[dirn].at[s],
                recv_sem=recv_sems[dirn].at[slot, s],
                device_id=(peer[dirn],),
                device_id_type=pl.DeviceIdType.MESH,
            )

        inflight = {}
        for s in range(N_SUB):
            for dirn in (0, 1):
                d = mk(dirn, 0, s)
                d.start()
                inflight[(dirn, s)] = d

        for h in range(n_hops):
            for s in range(N_SUB):
                for dirn in (0, 1):
                    d = inflight[(dirn, s)]
                    d.wait_recv()
                    if h < N_DEV - 1:
                        slot = h % 2
                        rows = pl.ds(recv_idx_of(dirn, h) * chunk + s * sub, sub)
                        acc = (out_ref[rows, cols[dirn]]
                               + recv_buf[dirn][slot, pl.ds(s * sub, sub), :])
                        if h == N_DEV - 2:
                            acc = _gelu(acc)
                        out_ref[rows, cols[dirn]] = acc
                    if h <= n_hops - 3:
                        pl.semaphore_signal(
                            credits[dirn].at[s], inc=1,
                            device_id=(upstream[dirn],),
                            device_id_type=pl.DeviceIdType.MESH,
                        )
                    if h + 1 < n_hops:
                        if h + 1 >= 2:
                            pl.semaphore_wait(credits[dirn].at[s], 1)
                        d.wait_send()
                        d2 = mk(dirn, h + 1, s)
                        d2.start()
                        inflight[(dirn, s)] = d2
                    else:
                        d.wait_send()

    return pl.pallas_call(
        body,
        out_shape=jax.ShapeDtypeStruct((m, n), jnp.float32),
        in_specs=[
            pl.BlockSpec(memory_space=pltpu.VMEM),
            pl.BlockSpec(memory_space=pltpu.VMEM),
        ],
        out_specs=pl.BlockSpec(memory_space=pltpu.VMEM),
        scratch_shapes=[
            pltpu.VMEM((2, chunk, half), jnp.float32),
            pltpu.VMEM((2, chunk, half), jnp.float32),
            pltpu.SemaphoreType.DMA((N_SUB,)),
            pltpu.SemaphoreType.DMA((2, N_SUB)),
            pltpu.SemaphoreType.DMA((N_SUB,)),
            pltpu.SemaphoreType.DMA((2, N_SUB)),
            pltpu.SemaphoreType.REGULAR((N_SUB,)),
            pltpu.SemaphoreType.REGULAR((N_SUB,)),
        ],
        compiler_params=pltpu.CompilerParams(collective_id=0),
    )(x, w_mat)


# device time: 41904 ns/iter; 3.2431x vs baseline; 1.2203x over previous
import jax
import jax.numpy as jnp
from jax import lax
from jax.experimental import pallas as pl
from jax.experimental.pallas import tpu as pltpu

N_DEV = 8
M = 1024

PARTS = ((0, 384, (0, 1, 2)), (384, 384, (1, 2, 0)), (768, 256, (2, 0, 1)))

SCR = {"1a": (0, 256), "1b": (256, 256), "2a": (512, 128),
       "2b": (640, 128), "3": (768, 128)}
SCR_ROWS = 896


def _gelu(y):
    c = 0.7978845608028654
    return 0.5 * y * (1.0 + jnp.tanh(c * (y + 0.044715 * y * y * y)))


def kernel(x, w_mat):
    m, k = x.shape
    _, n = w_mat.shape

    def body(x_ref, w_ref, out_ref, scr_a, scr_b, scr_c,
             send_sems, recv_sems):
        my = lax.axis_index("i")
        cz = lax.div(my, 4)
        kk = lax.rem(my, 4)
        cx = jnp.where((kk == 1) | (kk == 2), 1, 0)
        cy = jnp.where(kk >= 2, 1, 0)

        def pos(cx_, cy_, cz_):
            return cz_ * 4 + 2 * cy_ + lax.rem(cx_ + cy_, 2)

        coord = (cx, cy, cz)
        partner = (pos(1 - cx, cy, cz), pos(cx, 1 - cy, cz),
                   pos(cx, cy, 1 - cz))

        barrier = pltpu.get_barrier_semaphore()
        for d in range(3):
            pl.semaphore_signal(
                barrier, inc=1,
                device_id=(partner[d],), device_id_type=pl.DeviceIdType.MESH,
            )
        pl.semaphore_wait(barrier, 3)

        out_ref[:, :] = jnp.dot(
            x_ref[:, :], w_ref[:, :], preferred_element_type=jnp.float32
        )

        scratch = (scr_a, scr_b, scr_c)

        class Part:
            pass

        parts = []
        for pi, (col0, w, dims) in enumerate(PARTS):
            P = Part()
            P.pi = pi
            P.cols = pl.ds(col0, w)
            P.scr = scratch[pi]
            P.peer = tuple(partner[d] for d in dims)
            cd1, cd2, cd3 = (coord[d] for d in dims)
            P.keep1 = cd1 * 512
            P.send1 = (1 - cd1) * 512
            P.k2 = P.keep1 + cd2 * 256
            P.q2s = P.keep1 + (1 - cd2) * 256
            P.o3 = P.k2 + cd3 * 128
            P.r1 = P.k2 + (1 - cd3) * 128
            P.cd = (cd1, cd2, cd3)
            P.last_send = {0: None, 1: None}
            P.nsend = 0
            parts.append(P)

        recv_idx = {"1a": 0, "1b": 1, "2a": 2, "2b": 3, "3": 4,
                    "ag1": 5, "ag2a": 6, "ag2b": 7, "ag3a": 8, "ag3b": 9}

        def start(P, src_rows, nrows, peer_id, tag, dst_rows=None):
            src = out_ref.at[pl.ds(src_rows, nrows), P.cols]
            if dst_rows is not None:
                dst = P.scr.at[pl.ds(dst_rows, nrows), :]
            else:
                dst = out_ref.at[pl.ds(src_rows, nrows), P.cols]
            sem = P.nsend % 2
            rdma = pltpu.make_async_remote_copy(
                src_ref=src,
                dst_ref=dst,
                send_sem=send_sems.at[P.pi, sem],
                recv_sem=recv_sems.at[P.pi, recv_idx[tag]],
                device_id=(peer_id,),
                device_id_type=pl.DeviceIdType.MESH,
            )
            if P.last_send[sem] is not None:
                P.last_send[sem].wait_send()
            rdma.start()
            P.last_send[sem] = rdma
            P.nsend += 1
            P.desc = getattr(P, "desc", {})
            P.desc[tag] = rdma
            return rdma

        def add_from_scratch(P, rows, nrows, tag):
            base, _ = SCR[tag]
            sl = pl.ds(rows, nrows)
            out_ref[sl, P.cols] = (
                out_ref[sl, P.cols] + P.scr[pl.ds(base, nrows), :]
            )

        for P in parts:
            cd1, cd2, cd3 = P.cd
            P.start_1a = P.send1 + (1 - cd2) * 256
            P.start_1b = P.send1 + cd2 * 256
            start(P, P.start_1a, 256, P.peer[0], "1a", dst_rows=SCR["1a"][0])
            start(P, P.start_1b, 256, P.peer[0], "1b", dst_rows=SCR["1b"][0])

        for P in parts:
            cd1, cd2, cd3 = P.cd
            P.desc["1a"].wait_recv()
            add_from_scratch(P, P.keep1 + (1 - cd2) * 256, 256, "1a")
            start(P, P.q2s + (1 - cd3) * 128, 128, P.peer[1], "2a",
                  dst_rows=SCR["2a"][0])
            start(P, P.q2s + cd3 * 128, 128, P.peer[1], "2b",
                  dst_rows=SCR["2b"][0])

        for P in parts:
            cd1, cd2, cd3 = P.cd
            P.desc["1b"].wait_recv()
            add_from_scratch(P, P.keep1 + cd2 * 256, 256, "1b")

        for P in parts:
            cd1, cd2, cd3 = P.cd
            P.desc["2a"].wait_recv()
            add_from_scratch(P, P.k2 + (1 - cd3) * 128, 128, "2a")
            start(P, P.k2 + (1 - cd3) * 128, 128, P.peer[2], "3",
                  dst_rows=SCR["3"][0])

        for P in parts:
            cd1, cd2, cd3 = P.cd
            P.desc["2b"].wait_recv()
            add_from_scratch(P, P.k2 + cd3 * 128, 128, "2b")

        for P in parts:
            P.desc["3"].wait_recv()
            base, _ = SCR["3"]
            sl = pl.ds(P.o3, 128)
            P_final = out_ref[sl, P.cols] + P.scr[pl.ds(base, 128), :]
            out_ref[sl, P.cols] = _gelu(P_final)
            start(P, P.o3, 128, P.peer[2], "ag1")
            start(P, P.o3, 128, P.peer[1], "ag2a")

        for P in parts:
            P.desc["ag1"].wait_recv()
            start(P, P.r1, 128, P.peer[1], "ag2b")
            start(P, P.k2, 256, P.peer[0], "ag3a")

        for P in parts:
            cd1, cd2, cd3 = P.cd
            P.desc["ag2a"].wait_recv()
            P.desc["ag2b"].wait_recv()
            start(P, P.keep1 + (1 - cd2) * 256, 256, P.peer[0], "ag3b")

        for P in parts:
            P.desc["ag3a"].wait_recv()
            P.desc["ag3b"].wait_recv()

        for P in parts:
            for sem in (0, 1):
                if P.last_send[sem] is not None:
                    P.last_send[sem].wait_send()

    return pl.pallas_call(
        body,
        out_shape=jax.ShapeDtypeStruct((m, n), jnp.float32),
        in_specs=[
            pl.BlockSpec(memory_space=pltpu.VMEM),
            pl.BlockSpec(memory_space=pltpu.VMEM),
        ],
        out_specs=pl.BlockSpec(memory_space=pltpu.VMEM),
        scratch_shapes=[
            pltpu.VMEM((SCR_ROWS, PARTS[0][1]), jnp.float32),
            pltpu.VMEM((SCR_ROWS, PARTS[1][1]), jnp.float32),
            pltpu.VMEM((SCR_ROWS, PARTS[2][1]), jnp.float32),
            pltpu.SemaphoreType.DMA((3, 2)),
            pltpu.SemaphoreType.DMA((3, 10)),
        ],
        compiler_params=pltpu.CompilerParams(collective_id=0),
    )(x, w_mat)


# device time: 40415 ns/iter; 3.3626x vs baseline; 1.0368x over previous
import jax
import jax.numpy as jnp
from jax import lax
from jax.experimental import pallas as pl
from jax.experimental.pallas import tpu as pltpu

N_DEV = 8

PARTS = ((0, 384, (0, 1, 2)), (384, 384, (1, 2, 0)), (768, 256, (2, 0, 1)))

SCR = {"u1": 0, "u2": 128, "u3": 256, "u4": 384,
       "2a": 512, "2b": 640, "3": 768}
SCR_ROWS = 896

RECV_TAGS = ("u1", "u2", "u3", "u4", "2a", "2b", "3",
             "ag1", "ag2a", "ag2b", "ag31", "ag32", "ag33", "ag34")
N_SEND_SEMS = 4


def _gelu(y):
    c = 0.7978845608028654
    return 0.5 * y * (1.0 + jnp.tanh(c * (y + 0.044715 * y * y * y)))


def kernel(x, w_mat):
    m, k = x.shape
    _, n = w_mat.shape

    def body(x_ref, w_ref, out_ref, scr_a, scr_b, scr_c,
             send_sems, recv_sems):
        my = lax.axis_index("i")
        cz = lax.div(my, 4)
        kk = lax.rem(my, 4)
        cx = jnp.where((kk == 1) | (kk == 2), 1, 0)
        cy = jnp.where(kk >= 2, 1, 0)

        def pos(cx_, cy_, cz_):
            return cz_ * 4 + 2 * cy_ + lax.rem(cx_ + cy_, 2)

        coord = (cx, cy, cz)
        partner = (pos(1 - cx, cy, cz), pos(cx, 1 - cy, cz),
                   pos(cx, cy, 1 - cz))

        barrier = pltpu.get_barrier_semaphore()
        for d in range(3):
            pl.semaphore_signal(
                barrier, inc=1,
                device_id=(partner[d],), device_id_type=pl.DeviceIdType.MESH,
            )

        scratch = (scr_a, scr_b, scr_c)

        class Part:
            pass

        parts = []
        for pi, (col0, w, dims) in enumerate(PARTS):
            P = Part()
            P.pi = pi
            P.col0, P.w = col0, w
            P.cols = pl.ds(col0, w)
            P.scr = scratch[pi]
            P.peer = tuple(partner[d] for d in dims)
            cd1, cd2, cd3 = (coord[d] for d in dims)
            P.cd = (cd1, cd2, cd3)
            P.keep1 = cd1 * 512
            P.send1 = (1 - cd1) * 512
            P.k2 = P.keep1 + cd2 * 256
            P.q2s = P.keep1 + (1 - cd2) * 256
            P.o3 = P.k2 + cd3 * 128
            P.r1 = P.k2 + (1 - cd3) * 128
            P.last_send = [None] * N_SEND_SEMS
            P.nsend = 0
            P.desc = {}
            parts.append(P)

        def start(P, src_rows, nrows, peer_id, tag, scr_tag=None):
            src = out_ref.at[pl.ds(src_rows, nrows), P.cols]
            if scr_tag is not None:
                dst = P.scr.at[pl.ds(SCR[scr_tag], nrows), :]
            else:
                dst = out_ref.at[pl.ds(src_rows, nrows), P.cols]
            sem = P.nsend % N_SEND_SEMS
            rdma = pltpu.make_async_remote_copy(
                src_ref=src,
                dst_ref=dst,
                send_sem=send_sems.at[P.pi, sem],
                recv_sem=recv_sems.at[P.pi, RECV_TAGS.index(tag)],
                device_id=(peer_id,),
                device_id_type=pl.DeviceIdType.MESH,
            )
            if P.last_send[sem] is not None:
                P.last_send[sem].wait_send()
            rdma.start()
            P.last_send[sem] = rdma
            P.nsend += 1
            P.desc[tag] = rdma

        def add(P, rows, scr_tag):
            sl = pl.ds(rows, 128)
            out_ref[sl, P.cols] = (
                out_ref[sl, P.cols] + P.scr[pl.ds(SCR[scr_tag], 128), :]
            )

        for P in parts:
            cd1, cd2, cd3 = P.cd
            out_ref[:, P.cols] = jnp.dot(
                x_ref[:, :], w_ref[:, pl.ds(P.col0, P.w)],
                preferred_element_type=jnp.float32,
            )
            if P.pi == 0:
                pl.semaphore_wait(barrier, 3)
            start(P, P.send1 + (1 - cd2) * 256 + (1 - cd3) * 128, 128,
                  P.peer[0], "u1", scr_tag="u1")
            start(P, P.send1 + (1 - cd2) * 256 + cd3 * 128, 128,
                  P.peer[0], "u2", scr_tag="u2")
            start(P, P.send1 + cd2 * 256 + (1 - cd3) * 128, 128,
                  P.peer[0], "u3", scr_tag="u3")
            start(P, P.send1 + cd2 * 256 + cd3 * 128, 128,
                  P.peer[0], "u4", scr_tag="u4")

        for P in parts:
            cd1, cd2, cd3 = P.cd
            P.desc["u1"].wait_recv()
            add(P, P.keep1 + (1 - cd2) * 256 + (1 - cd3) * 128, "u1")
            start(P, P.q2s + (1 - cd3) * 128, 128, P.peer[1], "2a",
                  scr_tag="2a")

        for P in parts:
            cd1, cd2, cd3 = P.cd
            P.desc["u2"].wait_recv()
            add(P, P.keep1 + (1 - cd2) * 256 + cd3 * 128, "u2")
            start(P, P.q2s + cd3 * 128, 128, P.peer[1], "2b", scr_tag="2b")

        for P in parts:
            cd1, cd2, cd3 = P.cd
            P.desc["u3"].wait_recv()
            add(P, P.keep1 + cd2 * 256 + (1 - cd3) * 128, "u3")

        for P in parts:
            cd1, cd2, cd3 = P.cd
            P.desc["2a"].wait_recv()
            add(P, P.k2 + (1 - cd3) * 128, "2a")
            start(P, P.k2 + (1 - cd3) * 128, 128, P.peer[2], "3",
                  scr_tag="3")

        for P in parts:
            cd1, cd2, cd3 = P.cd
            P.desc["u4"].wait_recv()
            add(P, P.keep1 + cd2 * 256 + cd3 * 128, "u4")

        for P in parts:
            P.desc["2b"].wait_recv()
            add(P, P.k2 + P.cd[2] * 128, "2b")

        for P in parts:
            P.desc["3"].wait_recv()
            sl = pl.ds(P.o3, 128)
            final = out_ref[sl, P.cols] + P.scr[pl.ds(SCR["3"], 128), :]
            out_ref[sl, P.cols] = _gelu(final)
            start(P, P.o3, 128, P.peer[2], "ag1")
            start(P, P.o3, 128, P.peer[1], "ag2a")
            start(P, P.o3, 128, P.peer[0], "ag31")

        for P in parts:
            P.desc["ag1"].wait_recv()
            start(P, P.r1, 128, P.peer[1], "ag2b")
            start(P, P.r1, 128, P.peer[0], "ag32")

        for P in parts:
            cd1, cd2, cd3 = P.cd
            P.desc["ag2a"].wait_recv()
            start(P, P.keep1 + (1 - cd2) * 256 + cd3 * 128, 128,
                  P.peer[0], "ag33")

        for P in parts:
            cd1, cd2, cd3 = P.cd
            P.desc["ag2b"].wait_recv()
            start(P, P.keep1 + (1 - cd2) * 256 + (1 - cd3) * 128, 128,
                  P.peer[0], "ag34")

        for P in parts:
            for tag in ("ag31", "ag32", "ag33", "ag34"):
                P.desc[tag].wait_recv()

        for P in parts:
            for d in P.last_send:
                if d is not None:
                    d.wait_send()

    return pl.pallas_call(
        body,
        out_shape=jax.ShapeDtypeStruct((m, n), jnp.float32),
        in_specs=[
            pl.BlockSpec(memory_space=pltpu.VMEM),
            pl.BlockSpec(memory_space=pltpu.VMEM),
        ],
        out_specs=pl.BlockSpec(memory_space=pltpu.VMEM),
        scratch_shapes=[
            pltpu.VMEM((SCR_ROWS, PARTS[0][1]), jnp.float32),
            pltpu.VMEM((SCR_ROWS, PARTS[1][1]), jnp.float32),
            pltpu.VMEM((SCR_ROWS, PARTS[2][1]), jnp.float32),
            pltpu.SemaphoreType.DMA((3, N_SEND_SEMS)),
            pltpu.SemaphoreType.DMA((3, len(RECV_TAGS))),
        ],
        compiler_params=pltpu.CompilerParams(collective_id=0),
    )(x, w_mat)


# device time: 39839 ns/iter; 3.4112x vs baseline; 1.0145x over previous
import jax
import jax.numpy as jnp
from jax import lax
from jax.experimental import pallas as pl
from jax.experimental.pallas import tpu as pltpu

N_DEV = 8

PARTS = ((0, 384, (0, 2, 1)), (384, 384, (1, 0, 2)), (768, 256, (2, 1, 0)))

SCR = {"u1": 0, "u2": 128, "u3": 256, "u4": 384,
       "2a": 512, "2b": 640, "3": 768}
SCR_ROWS = 896

RECV_TAGS = ("u1", "u2", "u3", "u4", "2a", "2b", "3",
             "ag1", "ag2a", "ag2b", "ag31", "ag32", "ag33", "ag34")
N_SEND_SEMS = 4


def _gelu(y):
    c = 0.7978845608028654
    return 0.5 * y * (1.0 + jnp.tanh(c * (y + 0.044715 * y * y * y)))


def kernel(x, w_mat):
    m, k = x.shape
    _, n = w_mat.shape

    def body(x_ref, w_ref, out_ref, scr_a, scr_b, scr_c,
             send_sems, recv_sems):
        my = lax.axis_index("i")
        cz = lax.div(my, 4)
        kk = lax.rem(my, 4)
        cx = jnp.where((kk == 1) | (kk == 2), 1, 0)
        cy = jnp.where(kk >= 2, 1, 0)

        def pos(cx_, cy_, cz_):
            return cz_ * 4 + 2 * cy_ + lax.rem(cx_ + cy_, 2)

        coord = (cx, cy, cz)
        partner = (pos(1 - cx, cy, cz), pos(cx, 1 - cy, cz),
                   pos(cx, cy, 1 - cz))

        barrier = pltpu.get_barrier_semaphore()
        for d in range(3):
            pl.semaphore_signal(
                barrier, inc=1,
                device_id=(partner[d],), device_id_type=pl.DeviceIdType.MESH,
            )

        scratch = (scr_a, scr_b, scr_c)

        class Part:
            pass

        parts = []
        for pi, (col0, w, dims) in enumerate(PARTS):
            P = Part()
            P.pi = pi
            P.col0, P.w = col0, w
            P.cols = pl.ds(col0, w)
            P.scr = scratch[pi]
            P.peer = tuple(partner[d] for d in dims)
            cd1, cd2, cd3 = (coord[d] for d in dims)
            P.cd = (cd1, cd2, cd3)
            P.keep1 = cd1 * 512
            P.send1 = (1 - cd1) * 512
            P.k2 = P.keep1 + cd2 * 256
            P.q2s = P.keep1 + (1 - cd2) * 256
            P.o3 = P.k2 + cd3 * 128
            P.r1 = P.k2 + (1 - cd3) * 128
            P.last_send = [None] * N_SEND_SEMS
            P.nsend = 0
            P.desc = {}
            parts.append(P)

        def start(P, src_rows, nrows, peer_id, tag, scr_tag=None):
            src = out_ref.at[pl.ds(src_rows, nrows), P.cols]
            if scr_tag is not None:
                dst = P.scr.at[pl.ds(SCR[scr_tag], nrows), :]
            else:
                dst = out_ref.at[pl.ds(src_rows, nrows), P.cols]
            sem = P.nsend % N_SEND_SEMS
            rdma = pltpu.make_async_remote_copy(
                src_ref=src,
                dst_ref=dst,
                send_sem=send_sems.at[P.pi, sem],
                recv_sem=recv_sems.at[P.pi, RECV_TAGS.index(tag)],
                device_id=(peer_id,),
                device_id_type=pl.DeviceIdType.MESH,
            )
            if P.last_send[sem] is not None:
                P.last_send[sem].wait_send()
            rdma.start()
            P.last_send[sem] = rdma
            P.nsend += 1
            P.desc[tag] = rdma

        def add(P, rows, scr_tag):
            sl = pl.ds(rows, 128)
            out_ref[sl, P.cols] = (
                out_ref[sl, P.cols] + P.scr[pl.ds(SCR[scr_tag], 128), :]
            )

        for P in parts:
            cd1, cd2, cd3 = P.cd
            out_ref[:, P.cols] = jnp.dot(
                x_ref[:, :], w_ref[:, pl.ds(P.col0, P.w)],
                preferred_element_type=jnp.float32,
            )
            if P.pi == 0:
                pl.semaphore_wait(barrier, 3)
            start(P, P.send1 + (1 - cd2) * 256 + (1 - cd3) * 128, 128,
                  P.peer[0], "u1", scr_tag="u1")
            start(P, P.send1 + (1 - cd2) * 256 + cd3 * 128, 128,
                  P.peer[0], "u2", scr_tag="u2")
            start(P, P.send1 + cd2 * 256 + (1 - cd3) * 128, 128,
                  P.peer[0], "u3", scr_tag="u3")
            start(P, P.send1 + cd2 * 256 + cd3 * 128, 128,
                  P.peer[0], "u4", scr_tag="u4")

        for P in parts:
            cd1, cd2, cd3 = P.cd
            P.desc["u1"].wait_recv()
            add(P, P.keep1 + (1 - cd2) * 256 + (1 - cd3) * 128, "u1")
            start(P, P.q2s + (1 - cd3) * 128, 128, P.peer[1], "2a",
                  scr_tag="2a")

        for P in parts:
            cd1, cd2, cd3 = P.cd
            P.desc["u2"].wait_recv()
            add(P, P.keep1 + (1 - cd2) * 256 + cd3 * 128, "u2")
            start(P, P.q2s + cd3 * 128, 128, P.peer[1], "2b", scr_tag="2b")

        for P in parts:
            cd1, cd2, cd3 = P.cd
            P.desc["u3"].wait_recv()
            add(P, P.keep1 + cd2 * 256 + (1 - cd3) * 128, "u3")

        for P in parts:
            cd1, cd2, cd3 = P.cd
            P.desc["2a"].wait_recv()
            add(P, P.k2 + (1 - cd3) * 128, "2a")
            start(P, P.k2 + (1 - cd3) * 128, 128, P.peer[2], "3",
                  scr_tag="3")

        for P in parts:
            cd1, cd2, cd3 = P.cd
            P.desc["u4"].wait_recv()
            add(P, P.keep1 + cd2 * 256 + cd3 * 128, "u4")

        for P in parts:
            P.desc["2b"].wait_recv()
            add(P, P.k2 + P.cd[2] * 128, "2b")

        for P in parts:
            P.desc["3"].wait_recv()
            sl = pl.ds(P.o3, 128)
            final = out_ref[sl, P.cols] + P.scr[pl.ds(SCR["3"], 128), :]
            out_ref[sl, P.cols] = _gelu(final)
            start(P, P.o3, 128, P.peer[2], "ag1")
            start(P, P.o3, 128, P.peer[1], "ag2a")
            start(P, P.o3, 128, P.peer[0], "ag31")

        for P in parts:
            P.desc["ag1"].wait_recv()
            start(P, P.r1, 128, P.peer[1], "ag2b")
            start(P, P.r1, 128, P.peer[0], "ag32")

        for P in parts:
            cd1, cd2, cd3 = P.cd
            P.desc["ag2a"].wait_recv()
            start(P, P.keep1 + (1 - cd2) * 256 + cd3 * 128, 128,
                  P.peer[0], "ag33")

        for P in parts:
            cd1, cd2, cd3 = P.cd
            P.desc["ag2b"].wait_recv()
            start(P, P.keep1 + (1 - cd2) * 256 + (1 - cd3) * 128, 128,
                  P.peer[0], "ag34")

        for P in parts:
            for tag in ("ag31", "ag32", "ag33", "ag34"):
                P.desc[tag].wait_recv()

        for P in parts:
            for d in P.last_send:
                if d is not None:
                    d.wait_send()

    return pl.pallas_call(
        body,
        out_shape=jax.ShapeDtypeStruct((m, n), jnp.float32),
        in_specs=[
            pl.BlockSpec(memory_space=pltpu.VMEM),
            pl.BlockSpec(memory_space=pltpu.VMEM),
        ],
        out_specs=pl.BlockSpec(memory_space=pltpu.VMEM),
        scratch_shapes=[
            pltpu.VMEM((SCR_ROWS, PARTS[0][1]), jnp.float32),
            pltpu.VMEM((SCR_ROWS, PARTS[1][1]), jnp.float32),
            pltpu.VMEM((SCR_ROWS, PARTS[2][1]), jnp.float32),
            pltpu.SemaphoreType.DMA((3, N_SEND_SEMS)),
            pltpu.SemaphoreType.DMA((3, len(RECV_TAGS))),
        ],
        compiler_params=pltpu.CompilerParams(collective_id=0),
    )(x, w_mat)
